# baseline (speedup 1.0000x reference)
import jax
import jax.numpy as jnp
from jax import lax
from jax.experimental import pallas as pl
from jax.experimental.pallas import tpu as pltpu
from jax.experimental.pallas import tpu_sc as plsc

VOCAB = 50257
D = 128
NC = 2
NS = 16
NW = NC * NS

B = 16384 * 200
CH = 128
SC2 = 2 * CH
NCHUNK = B // CH
CPW = NCHUNK // NW
IK = 80
NGROUP = CPW // IK
NSC = IK // 2
NBUF = 2


def _body(tok_hbm, tab_hbm, out_hbm,
          idx0, idx1, r0, r1,
          g0, g1, s0, s1, i0, i1):
    rows = (r0, r1)
    idxb = (idx0, idx1)
    gsem = (g0, g1)
    ssem = (s0, s1)
    isem = (i0, i1)

    wid = lax.axis_index("s") * NC + lax.axis_index("c")
    chunk0 = wid * CPW

    def gwait(b):
        pltpu.make_async_copy(tab_hbm.at[pl.ds(0, SC2)], rows[b], gsem[b]).wait()

    def swait(b):
        pltpu.make_async_copy(rows[b], out_hbm.at[pl.ds(0, SC2)], ssem[b]).wait()

    def gpair(b, ib, s):
        pltpu.async_copy(tab_hbm.at[ib.at[2 * s]],
                         rows[b].at[pl.ds(0, CH)], gsem[b])
        pltpu.async_copy(tab_hbm.at[ib.at[2 * s + 1]],
                         rows[b].at[pl.ds(CH, CH)], gsem[b])

    def sstart(b, gchunk):
        pltpu.async_copy(rows[b], out_hbm.at[pl.ds(gchunk * CH, SC2)], ssem[b])

    pltpu.sync_copy(tok_hbm.at[pl.ds(chunk0, IK)], idxb[0])

    for G in range(NGROUP):
        ib = idxb[G % 2]
        gc0 = chunk0 + G * IK
        if G > 0:
            pltpu.make_async_copy(tok_hbm.at[pl.ds(0, IK)], ib, isem[G % 2]).wait()
        if G + 1 < NGROUP:
            pltpu.async_copy(tok_hbm.at[pl.ds(gc0 + IK, IK)],
                             idxb[(G + 1) % 2], isem[(G + 1) % 2])

        if G > 0:
            swait(0)
        gpair(0, ib, 0)

        def step(st, carry):
            for b in range(NBUF):
                s = st * NBUF + b
                n = s + 1
                bn = (b + 1) % NBUF

                if G == 0:
                    @pl.when(jnp.logical_and(n < NSC, n >= NBUF))
                    def _():
                        swait(bn)
                        gpair(bn, ib, n)

                    @pl.when(jnp.logical_and(n < NSC, n < NBUF))
                    def _():
                        gpair(bn, ib, n)
                else:
                    @pl.when(n < NSC)
                    def _():
                        swait(bn)
                        gpair(bn, ib, n)

                gwait(b)
                sstart(b, gc0 + 2 * s)
            return carry

        lax.fori_loop(0, NSC // NBUF, step, 0)

    for b in range(NBUF):
        swait(b)


@jax.jit
def _embed(tokens2d, table):
    kern = pl.kernel(
        _body,
        out_type=jax.ShapeDtypeStruct((B, D), jnp.float32),
        mesh=plsc.VectorSubcoreMesh(
            core_axis_name="c", subcore_axis_name="s",
            num_cores=NC, num_subcores=NS),
        scratch_types=[
            pltpu.VMEM((IK, CH), jnp.int32),
            pltpu.VMEM((IK, CH), jnp.int32),
        ] + [pltpu.VMEM((SC2, D), jnp.float32)] * NBUF
          + [pltpu.SemaphoreType.DMA] * (2 * NBUF + 2),
    )
    return kern(tokens2d, table)


def kernel(tokens, table):
    tokens2d = tokens.reshape(NCHUNK, CH).astype(jnp.int32)
    out = _embed(tokens2d, table)
    return out.reshape(tokens.shape + (D,))
